# Initial kernel scaffold; baseline (speedup 1.0000x reference)
import jax
import jax.numpy as jnp
from jax.experimental import pallas as pl


def kernel(local_coords, atom_types, bb_num_vec, emb, edge_w1, edge_b1, edge_w2, edge_b2, node_w1, node_b1, node_w2, node_b2, coord_w1, coord_b1, coord_w2):
    raise NotImplementedError("write your pallas kernel here")


# speedup vs baseline: 9.6920x; 9.6920x over previous
import functools

import jax
import jax.numpy as jnp
from jax import lax
from jax.experimental import pallas as pl
from jax.experimental.pallas import tpu as pltpu

P = 32
G = 4
MAX_RADIUS = 2.0
ANG_TO_NM = 0.1
RF = 24


def _egnn_body(L, BLOCK, DE, HID, D, coeff,
               xi_ref, xj_ref, aidx_ref, emb_ref,
               w1a_ref, w1b_ref, w1c_ref, b1_ref, w2_ref, b2_ref,
               na_ref, nb_ref, nb1_ref, nw2_ref, nb2_ref,
               out_ref):
    f32 = jnp.float32
    NG = G * P
    E = G * P * P

    xi = xi_ref[...]
    xj = xj_ref[...]
    diff = xi - xj
    d2 = jnp.sum(diff * diff, axis=3, keepdims=True)

    ii = lax.broadcasted_iota(jnp.int32, (G, P, P, 1), 1)
    jj = lax.broadcasted_iota(jnp.int32, (G, P, P, 1), 2)
    em = ((d2 < MAX_RADIUS * MAX_RADIUS) & (ii != jj)
          & (jj < BLOCK)).astype(f32)

    dist = jnp.sqrt(d2 + 1e-12)
    off = (MAX_RADIUS / (DE - 1)) * lax.broadcasted_iota(
        jnp.int32, (1, 1, 1, DE), 3).astype(f32)
    ea = jnp.exp(coeff * (dist - off) ** 2)
    radial = (ANG_TO_NM * ANG_TO_NM) * d2
    pad = jnp.zeros((G, P, P, RF - 1 - DE), f32)
    rflat = jnp.concatenate([radial, ea, pad], axis=3).reshape(E, RF)

    aidx = aidx_ref[...]
    oneh = (aidx == lax.broadcasted_iota(jnp.int32, (1, 128), 1)).astype(f32)
    h = jnp.dot(oneh, emb_ref[...], preferred_element_type=f32)

    for l in range(L):
        hr = jnp.dot(h, w1a_ref[l], preferred_element_type=f32)
        hc = jnp.dot(h, w1b_ref[l], preferred_element_type=f32)
        rp = jnp.dot(rflat, w1c_ref[l], preferred_element_type=f32)
        m1 = (rp.reshape(G, P, P, HID)
              + jnp.broadcast_to(hr.reshape(G, P, 1, HID), (G, P, P, HID))
              + jnp.broadcast_to(hc.reshape(G, 1, P, HID), (G, P, P, HID))
              ).reshape(E, HID)
        m1 = jax.nn.relu(m1 + b1_ref[l:l + 1, :])
        m2 = jax.nn.relu(jnp.dot(m1, w2_ref[l], preferred_element_type=f32)
                         + b2_ref[l:l + 1, :])
        agg = (m2.reshape(G, P, P, HID) * em).sum(axis=2).reshape(NG, HID)
        nm = jax.nn.relu(jnp.dot(h, na_ref[l], preferred_element_type=f32)
                         + jnp.dot(agg, nb_ref[l], preferred_element_type=f32)
                         + nb1_ref[l:l + 1, :])
        nm = jnp.dot(nm, nw2_ref[l], preferred_element_type=f32) + nb2_ref[l:l + 1, :]
        h = 2.0 * h + nm

    pooled = h.reshape(G, P, D)[:, :BLOCK, :].sum(axis=1) * (1.0 / BLOCK)
    out_ref[...] = pooled.reshape(1, G, D)


def kernel(local_coords, atom_types, bb_num_vec, emb, edge_w1, edge_b1,
           edge_w2, edge_b2, node_w1, node_b1, node_w2, node_b2,
           coord_w1, coord_b1, coord_w2):
    f32 = jnp.float32
    N = local_coords.shape[0]
    NB = bb_num_vec.shape[0]
    BLOCK = N // NB
    D = emb.shape[1]
    HID = edge_w2.shape[1]
    L = edge_w1.shape[0]
    DE = edge_w1.shape[1] - 2 * D - 1
    coeff = -0.5 / (MAX_RADIUS / (DE - 1)) ** 2

    lc3 = local_coords.astype(f32).reshape(NB, BLOCK, 3)
    lcp = jnp.pad(lc3, ((0, 0), (0, P - BLOCK), (0, 0)))
    xi = lcp[:, :, None, :]
    xj = lcp[:, None, :, :]

    ai = (atom_types.astype(jnp.int32) - 1) % emb.shape[0]
    aip = jnp.pad(ai.reshape(NB, BLOCK), ((0, 0), (0, P - BLOCK)))
    aip = aip.reshape(NB * P, 1)

    embp = jnp.pad(emb.astype(f32), ((0, 128 - emb.shape[0]), (0, 0)))

    w1a = edge_w1[:, :D, :]
    w1b = edge_w1[:, D:2 * D, :]
    w1c = jnp.pad(edge_w1[:, 2 * D:, :], ((0, 0), (0, RF - 1 - DE), (0, 0)))
    na = node_w1[:, :D, :]
    nb = node_w1[:, D:, :]

    body = functools.partial(_egnn_body, L, BLOCK, DE, HID, D, coeff)
    out = pl.pallas_call(
        body,
        grid=(NB // G,),
        in_specs=[
            pl.BlockSpec((G, P, 1, 3), lambda b: (b, 0, 0, 0)),
            pl.BlockSpec((G, 1, P, 3), lambda b: (b, 0, 0, 0)),
            pl.BlockSpec((G * P, 1), lambda b: (b, 0)),
            pl.BlockSpec((128, 128), lambda b: (0, 0)),
            pl.BlockSpec((L, D, HID), lambda b: (0, 0, 0)),
            pl.BlockSpec((L, D, HID), lambda b: (0, 0, 0)),
            pl.BlockSpec((L, RF, HID), lambda b: (0, 0, 0)),
            pl.BlockSpec((L, HID), lambda b: (0, 0)),
            pl.BlockSpec((L, HID, HID), lambda b: (0, 0, 0)),
            pl.BlockSpec((L, HID), lambda b: (0, 0)),
            pl.BlockSpec((L, D, HID), lambda b: (0, 0, 0)),
            pl.BlockSpec((L, HID, HID), lambda b: (0, 0, 0)),
            pl.BlockSpec((L, HID), lambda b: (0, 0)),
            pl.BlockSpec((L, HID, D), lambda b: (0, 0, 0)),
            pl.BlockSpec((L, D), lambda b: (0, 0)),
        ],
        out_specs=pl.BlockSpec((1, G, D), lambda b: (b, 0, 0)),
        out_shape=jax.ShapeDtypeStruct((NB // G, G, D), f32),
        compiler_params=pltpu.CompilerParams(
            dimension_semantics=("arbitrary",)),
    )(xi, xj, aip, embp, w1a, w1b, w1c, edge_b1, edge_w2, edge_b2,
      na, nb, node_b1, node_w2, node_b2)
    return out.reshape(NB, D)


# speedup vs baseline: 10.1652x; 1.0488x over previous
import functools

import jax
import jax.numpy as jnp
from jax import lax
from jax.experimental import pallas as pl
from jax.experimental.pallas import tpu as pltpu

P = 32
G = 8
MAX_RADIUS = 2.0
ANG_TO_NM = 0.1
RF = 24


def _egnn_body(L, BLOCK, DE, HID, D, coeff,
               xi_ref, xj_ref, aidx_ref, emb_ref,
               w1a_ref, w1b_ref, w1c_ref, b1_ref, w2_ref, b2_ref,
               na_ref, nb_ref, nb1_ref, nw2_ref, nb2_ref,
               out_ref):
    f32 = jnp.float32
    NG = G * P
    E = G * P * P

    xi = xi_ref[...]
    xj = xj_ref[...]
    diff = xi - xj
    d2 = jnp.sum(diff * diff, axis=3, keepdims=True)

    ii = lax.broadcasted_iota(jnp.int32, (G, P, P, 1), 1)
    jj = lax.broadcasted_iota(jnp.int32, (G, P, P, 1), 2)
    em = ((d2 < MAX_RADIUS * MAX_RADIUS) & (ii != jj)
          & (jj < BLOCK)).astype(f32)

    dist = jnp.sqrt(d2 + 1e-12)
    off = (MAX_RADIUS / (DE - 1)) * lax.broadcasted_iota(
        jnp.int32, (1, 1, 1, DE), 3).astype(f32)
    ea = jnp.exp(coeff * (dist - off) ** 2)
    radial = (ANG_TO_NM * ANG_TO_NM) * d2
    pad = jnp.zeros((G, P, P, RF - 1 - DE), f32)
    rflat = jnp.concatenate([radial, ea, pad], axis=3).reshape(E, RF)

    aidx = aidx_ref[...]
    oneh = (aidx == lax.broadcasted_iota(jnp.int32, (1, 128), 1)).astype(f32)
    h = jnp.dot(oneh, emb_ref[...], preferred_element_type=f32)

    for l in range(L):
        hr = jnp.dot(h, w1a_ref[l], preferred_element_type=f32)
        hc = jnp.dot(h, w1b_ref[l], preferred_element_type=f32)
        rp = jnp.dot(rflat, w1c_ref[l], preferred_element_type=f32)
        m1 = (rp.reshape(G, P, P, HID)
              + jnp.broadcast_to(hr.reshape(G, P, 1, HID), (G, P, P, HID))
              + jnp.broadcast_to(hc.reshape(G, 1, P, HID), (G, P, P, HID))
              ).reshape(E, HID)
        m1 = jax.nn.relu(m1 + b1_ref[l:l + 1, :])
        m2 = jax.nn.relu(jnp.dot(m1, w2_ref[l], preferred_element_type=f32)
                         + b2_ref[l:l + 1, :])
        agg = (m2.reshape(G, P, P, HID) * em).sum(axis=2).reshape(NG, HID)
        nm = jax.nn.relu(jnp.dot(h, na_ref[l], preferred_element_type=f32)
                         + jnp.dot(agg, nb_ref[l], preferred_element_type=f32)
                         + nb1_ref[l:l + 1, :])
        nm = jnp.dot(nm, nw2_ref[l], preferred_element_type=f32) + nb2_ref[l:l + 1, :]
        h = 2.0 * h + nm

    pooled = h.reshape(G, P, D)[:, :BLOCK, :].sum(axis=1) * (1.0 / BLOCK)
    out_ref[...] = pooled.reshape(1, G, D)


def kernel(local_coords, atom_types, bb_num_vec, emb, edge_w1, edge_b1,
           edge_w2, edge_b2, node_w1, node_b1, node_w2, node_b2,
           coord_w1, coord_b1, coord_w2):
    f32 = jnp.float32
    N = local_coords.shape[0]
    NB = bb_num_vec.shape[0]
    BLOCK = N // NB
    D = emb.shape[1]
    HID = edge_w2.shape[1]
    L = edge_w1.shape[0]
    DE = edge_w1.shape[1] - 2 * D - 1
    coeff = -0.5 / (MAX_RADIUS / (DE - 1)) ** 2

    lc3 = local_coords.astype(f32).reshape(NB, BLOCK, 3)
    lcp = jnp.pad(lc3, ((0, 0), (0, P - BLOCK), (0, 0)))
    xi = lcp[:, :, None, :]
    xj = lcp[:, None, :, :]

    ai = (atom_types.astype(jnp.int32) - 1) % emb.shape[0]
    aip = jnp.pad(ai.reshape(NB, BLOCK), ((0, 0), (0, P - BLOCK)))
    aip = aip.reshape(NB * P, 1)

    embp = jnp.pad(emb.astype(f32), ((0, 128 - emb.shape[0]), (0, 0)))

    w1a = edge_w1[:, :D, :]
    w1b = edge_w1[:, D:2 * D, :]
    w1c = jnp.pad(edge_w1[:, 2 * D:, :], ((0, 0), (0, RF - 1 - DE), (0, 0)))
    na = node_w1[:, :D, :]
    nb = node_w1[:, D:, :]

    body = functools.partial(_egnn_body, L, BLOCK, DE, HID, D, coeff)
    out = pl.pallas_call(
        body,
        grid=(NB // G,),
        in_specs=[
            pl.BlockSpec((G, P, 1, 3), lambda b: (b, 0, 0, 0)),
            pl.BlockSpec((G, 1, P, 3), lambda b: (b, 0, 0, 0)),
            pl.BlockSpec((G * P, 1), lambda b: (b, 0)),
            pl.BlockSpec((128, 128), lambda b: (0, 0)),
            pl.BlockSpec((L, D, HID), lambda b: (0, 0, 0)),
            pl.BlockSpec((L, D, HID), lambda b: (0, 0, 0)),
            pl.BlockSpec((L, RF, HID), lambda b: (0, 0, 0)),
            pl.BlockSpec((L, HID), lambda b: (0, 0)),
            pl.BlockSpec((L, HID, HID), lambda b: (0, 0, 0)),
            pl.BlockSpec((L, HID), lambda b: (0, 0)),
            pl.BlockSpec((L, D, HID), lambda b: (0, 0, 0)),
            pl.BlockSpec((L, HID, HID), lambda b: (0, 0, 0)),
            pl.BlockSpec((L, HID), lambda b: (0, 0)),
            pl.BlockSpec((L, HID, D), lambda b: (0, 0, 0)),
            pl.BlockSpec((L, D), lambda b: (0, 0)),
        ],
        out_specs=pl.BlockSpec((1, G, D), lambda b: (b, 0, 0)),
        out_shape=jax.ShapeDtypeStruct((NB // G, G, D), f32),
        compiler_params=pltpu.CompilerParams(
            dimension_semantics=("arbitrary",)),
    )(xi, xj, aip, embp, w1a, w1b, w1c, edge_b1, edge_w2, edge_b2,
      na, nb, node_b1, node_w2, node_b2)
    return out.reshape(NB, D)


# speedup vs baseline: 21.0455x; 2.0703x over previous
import functools

import jax
import jax.numpy as jnp
from jax import lax
from jax.experimental import pallas as pl
from jax.experimental.pallas import tpu as pltpu

P = 32
G = 8
MAX_RADIUS = 2.0
ANG_TO_NM = 0.1
RF = 24


def _egnn_body(L, BLOCK, DE, HID, D, coeff,
               xi_ref, xj_ref, aidx_ref, emb_ref,
               w1a_ref, w1b_ref, w1c_ref, w2_ref, b2_ref,
               na_ref, nb_ref, nb1_ref, nw2_ref, nb2_ref,
               out_ref):
    f32 = jnp.float32
    NG = G * P
    E = G * P * P

    xi = xi_ref[...]
    xj = xj_ref[...]
    diff = xi - xj
    d2 = jnp.sum(diff * diff, axis=3, keepdims=True)

    ii = lax.broadcasted_iota(jnp.int32, (G, P, P, 1), 1)
    jj = lax.broadcasted_iota(jnp.int32, (G, P, P, 1), 2)
    em = ((d2 < MAX_RADIUS * MAX_RADIUS) & (ii != jj)
          & (jj < BLOCK)).astype(f32)

    dist = jnp.sqrt(d2 + 1e-12)
    off = (MAX_RADIUS / (DE - 1)) * lax.broadcasted_iota(
        jnp.int32, (1, 1, 1, DE), 3).astype(f32)
    ea = jnp.exp(coeff * (dist - off) ** 2)
    ones = jnp.ones((G, P, P, 1), f32)
    rfeat = jnp.concatenate([d2, ea, ones], axis=3).reshape(E, DE + 2)

    PP = P * P
    erow = lax.broadcasted_iota(jnp.int32, (PP, 2 * P), 0)
    ecol = lax.broadcasted_iota(jnp.int32, (PP, 2 * P), 1)
    ohij = (((erow // P) == ecol) | ((erow % P) == (ecol - P))).astype(f32)

    aidx = aidx_ref[...]
    oneh = (aidx == lax.broadcasted_iota(jnp.int32, (1, 128), 1)).astype(f32)
    h = jnp.dot(oneh, emb_ref[...], preferred_element_type=f32)

    for l in range(L):
        hr = jnp.dot(h, w1a_ref[l], preferred_element_type=f32)
        hc = jnp.dot(h, w1b_ref[l], preferred_element_type=f32)
        aggs = []
        for g in range(G):
            lhs = jnp.concatenate(
                [rfeat[g * PP:(g + 1) * PP, :], ohij], axis=1)
            rhs = jnp.concatenate(
                [w1c_ref[l], hr[g * P:(g + 1) * P, :],
                 hc[g * P:(g + 1) * P, :]], axis=0)
            m1 = jax.nn.relu(jnp.dot(lhs, rhs, preferred_element_type=f32))
            m2 = jax.nn.relu(jnp.dot(m1, w2_ref[l], preferred_element_type=f32)
                             + b2_ref[l:l + 1, :])
            m2 = m2 * em[g].reshape(PP, 1)
            aggs.append(m2.reshape(P, P, HID).sum(axis=1))
        agg = jnp.concatenate(aggs, axis=0)
        nm = jax.nn.relu(jnp.dot(h, na_ref[l], preferred_element_type=f32)
                         + jnp.dot(agg, nb_ref[l], preferred_element_type=f32)
                         + nb1_ref[l:l + 1, :])
        nm = jnp.dot(nm, nw2_ref[l], preferred_element_type=f32) + nb2_ref[l:l + 1, :]
        h = 2.0 * h + nm

    pooled = h.reshape(G, P, D)[:, :BLOCK, :].sum(axis=1) * (1.0 / BLOCK)
    out_ref[...] = pooled.reshape(1, G, D)


def kernel(local_coords, atom_types, bb_num_vec, emb, edge_w1, edge_b1,
           edge_w2, edge_b2, node_w1, node_b1, node_w2, node_b2,
           coord_w1, coord_b1, coord_w2):
    f32 = jnp.float32
    N = local_coords.shape[0]
    NB = bb_num_vec.shape[0]
    BLOCK = N // NB
    D = emb.shape[1]
    HID = edge_w2.shape[1]
    L = edge_w1.shape[0]
    DE = edge_w1.shape[1] - 2 * D - 1
    coeff = -0.5 / (MAX_RADIUS / (DE - 1)) ** 2

    lc3 = local_coords.astype(f32).reshape(NB, BLOCK, 3)
    lcp = jnp.pad(lc3, ((0, 0), (0, P - BLOCK), (0, 0)))
    xi = lcp[:, :, None, :]
    xj = lcp[:, None, :, :]

    ai = (atom_types.astype(jnp.int32) - 1) % emb.shape[0]
    aip = jnp.pad(ai.reshape(NB, BLOCK), ((0, 0), (0, P - BLOCK)))
    aip = aip.reshape(NB * P, 1)

    embp = jnp.pad(emb.astype(f32), ((0, 128 - emb.shape[0]), (0, 0)))

    w1a = edge_w1[:, :D, :]
    w1b = edge_w1[:, D:2 * D, :]
    w1c = jnp.concatenate([
        (ANG_TO_NM * ANG_TO_NM) * edge_w1[:, 2 * D:2 * D + 1, :],
        edge_w1[:, 2 * D + 1:, :],
        edge_b1[:, None, :],
    ], axis=1)
    na = node_w1[:, :D, :]
    nb = node_w1[:, D:, :]

    body = functools.partial(_egnn_body, L, BLOCK, DE, HID, D, coeff)
    out = pl.pallas_call(
        body,
        grid=(NB // G,),
        in_specs=[
            pl.BlockSpec((G, P, 1, 3), lambda b: (b, 0, 0, 0)),
            pl.BlockSpec((G, 1, P, 3), lambda b: (b, 0, 0, 0)),
            pl.BlockSpec((G * P, 1), lambda b: (b, 0)),
            pl.BlockSpec((128, 128), lambda b: (0, 0)),
            pl.BlockSpec((L, D, HID), lambda b: (0, 0, 0)),
            pl.BlockSpec((L, D, HID), lambda b: (0, 0, 0)),
            pl.BlockSpec((L, DE + 2, HID), lambda b: (0, 0, 0)),
            pl.BlockSpec((L, HID, HID), lambda b: (0, 0, 0)),
            pl.BlockSpec((L, HID), lambda b: (0, 0)),
            pl.BlockSpec((L, D, HID), lambda b: (0, 0, 0)),
            pl.BlockSpec((L, HID, HID), lambda b: (0, 0, 0)),
            pl.BlockSpec((L, HID), lambda b: (0, 0)),
            pl.BlockSpec((L, HID, D), lambda b: (0, 0, 0)),
            pl.BlockSpec((L, D), lambda b: (0, 0)),
        ],
        out_specs=pl.BlockSpec((1, G, D), lambda b: (b, 0, 0)),
        out_shape=jax.ShapeDtypeStruct((NB // G, G, D), f32),
        compiler_params=pltpu.CompilerParams(
            dimension_semantics=("arbitrary",)),
    )(xi, xj, aip, embp, w1a, w1b, w1c, edge_w2, edge_b2,
      na, nb, node_b1, node_w2, node_b2)
    return out.reshape(NB, D)
